# baseline (speedup 1.0000x reference)
import jax
import jax.numpy as jnp
from jax.experimental import pallas as pl
from jax.experimental.pallas import tpu as pltpu

K_RATIO = 0.5


def _score_body(h_ref, w_ref, b_ref, s_ref):
    hb = h_ref[0]
    w = w_ref[...]
    acc = jnp.dot(hb, w, preferred_element_type=jnp.float32)
    s_ref[0] = jax.nn.sigmoid(acc + b_ref[0])


def _scores(h, W, b):
    bs, n, d = h.shape
    blk = 2000
    nblk = n // blk
    w8 = jnp.pad(W, ((0, 0), (0, 7)))
    s3 = pl.pallas_call(
        _score_body,
        grid=(bs, nblk),
        in_specs=[
            pl.BlockSpec((1, blk, d), lambda i, j: (i, j, 0)),
            pl.BlockSpec((d, 8), lambda i, j: (0, 0)),
            pl.BlockSpec(memory_space=pltpu.SMEM),
        ],
        out_specs=pl.BlockSpec((1, blk, 8), lambda i, j: (i, j, 0)),
        out_shape=jax.ShapeDtypeStruct((bs, n, 8), jnp.float32),
    )(h, w8, b)
    return s3[:, :, 0]


def kernel(h, W, b):
    bs, n, d = h.shape
    n_keep = max(1, int(n * K_RATIO))
    s = _scores(h, W, b)
    _, top_idx = jax.lax.top_k(s, n_keep)
    hw = h * s[:, :, None]
    return jnp.take_along_axis(hw, top_idx[:, :, None], axis=1)


# speedup vs baseline: 1.2490x; 1.2490x over previous
import functools

import jax
import jax.numpy as jnp
from jax import lax
from jax.experimental import pallas as pl
from jax.experimental.pallas import tpu as pltpu
from jax.experimental.pallas import tpu_sc as plsc

K_RATIO = 0.5

B = 8
N = 50000
D = 128
K = 25000
NT = 4
C = 12544
CV = C // 16
CR = C // 128
N_PAD = NT * C

_RADIX = ((0, 2047), (11, 2047), (22, 1023))


def _score_body(h_ref, w_ref, b_ref, s_ref):
    hb = h_ref[0]
    w = w_ref[...]
    acc = jnp.dot(hb, w, preferred_element_type=jnp.float32)
    s_ref[0] = jax.nn.sigmoid(acc + b_ref[0])


def _scores(h, W, b):
    bs, n, d = h.shape
    blk = 2000
    s3 = pl.pallas_call(
        _score_body,
        grid=(bs, n // blk),
        in_specs=[
            pl.BlockSpec((1, blk, d), lambda i, j: (i, j, 0)),
            pl.BlockSpec((d, 8), lambda i, j: (0, 0)),
            pl.BlockSpec(memory_space=pltpu.SMEM),
        ],
        out_specs=pl.BlockSpec((1, blk, 8), lambda i, j: (i, j, 0)),
        out_shape=jax.ShapeDtypeStruct((bs, n, 8), jnp.float32),
    )(h, jnp.pad(W, ((0, 0), (0, 7))), b)
    return s3[:, :, 0]


def _dyn_gather(x, idx):
    dnums = lax.GatherDimensionNumbers(
        offset_dims=(), collapsed_slice_dims=(0,), start_index_map=(0,))
    return lax.gather(x, idx[:, None], dnums, (1,),
                      mode=lax.GatherScatterMode.PROMISE_IN_BOUNDS)


def _vreg_runs(d, lane):
    comp = d * 16 + lane
    csort, _ = plsc.sort_key_val(comp, lane)
    sd = csort >> 4
    ol = csort & 15
    prev = _dyn_gather(sd, jnp.maximum(lane - 1, 0))
    is_start = (lane == 0) | (sd != prev)
    startpos = plsc.cummax(jnp.where(is_start, lane, 0))
    rank = lane - startpos
    nxt = _dyn_gather(sd, jnp.minimum(lane + 1, 15))
    is_end = (lane == 15) | (sd != nxt)
    return sd, ol, rank, is_end


_sc_mesh = plsc.VectorSubcoreMesh(core_axis_name="c", subcore_axis_name="s")


@functools.partial(
    pl.kernel,
    out_type=jax.ShapeDtypeStruct((B * K, D), jnp.float32),
    mesh=_sc_mesh,
    scratch_types=[
        pltpu.VMEM((C,), jnp.int32),
        pltpu.VMEM((C,), jnp.int32),
        pltpu.VMEM((C,), jnp.int32),
        pltpu.VMEM((CR, 128), jnp.int32),
        pltpu.VMEM((2048,), jnp.int32),
        pltpu.VMEM((NT - 1, C), jnp.int32),
        pltpu.VMEM((98, 128), jnp.float32),
        pltpu.VMEM((1, 64), jnp.int32),
        pltpu.VMEM((1, 64), jnp.int32),
        pltpu.VMEM_SHARED((16, C), jnp.int32),
        pltpu.SemaphoreType.DMA,
    ],
    compiler_params=pltpu.CompilerParams(needs_layout_passes=False),
)
def _sc_topk(s_hbm, h_hbm, out_hbm, key_a, idx_a, key_b, idx_b, hist,
             peers, rows, idxc, rankc, shared, sem):
    ranks = key_a
    cid = lax.axis_index("c")
    sid = lax.axis_index("s")
    b = cid * 4 + sid // 4
    q = sid % 4
    g = sid // 4
    lane = lax.iota(jnp.int32, 16)
    q_v = jnp.full((16,), 0, jnp.int32) + q

    pltpu.sync_copy(s_hbm.at[b * 4 + q], rows.at[pl.ds(0, CR)])

    def _build(v, _):
        sv = rows[v >> 3, pl.ds((v & 7) * 16, 16)]
        key_a[pl.ds(v * 16, 16)] = ~lax.bitcast_convert_type(sv, jnp.int32)
        idx_a[pl.ds(v * 16, 16)] = q * C + v * 16 + lane
        return 0

    lax.fori_loop(0, CV, _build, 0)

    for p, (shift, mask) in enumerate(_RADIX):
        def _zero(i, _):
            hist[pl.ds(i * 16, 16)] = jnp.zeros((16,), jnp.int32)
            return 0

        lax.fori_loop(0, 128, _zero, 0)

        def _hist(v, _, shift=shift, mask=mask, src=(key_a, key_b)[p % 2]):
            k = src[pl.ds(v * 16, 16)]
            d = (k >> shift) & mask
            sd, _, rank, is_end = _vreg_runs(d, lane)
            plsc.addupdate_scatter(hist, [sd], rank + 1, mask=is_end)
            return 0

        lax.fori_loop(0, CV, _hist, 0)

        def _scan(i, run):
            hv = hist[pl.ds(i * 16, 16)]
            cs = plsc.cumsum(hv)
            hist[pl.ds(i * 16, 16)] = cs - hv + run
            return run + jnp.max(cs)

        lax.fori_loop(0, 128, _scan, jnp.int32(0))

        def _place(v, _, shift=shift, mask=mask, p=p):
            ksrc = (key_a, key_b)[p % 2]
            k = ksrc[pl.ds(v * 16, 16)]
            if p % 2 == 0:
                iv = idx_a[pl.ds(v * 16, 16)]
            else:
                iv = idx_b[v >> 3, pl.ds((v & 7) * 16, 16)]
            d = (k >> shift) & mask
            sd, ol, rank, is_end = _vreg_runs(d, lane)
            base = plsc.load_gather(hist, [sd])
            dest = base + rank
            k_s = _dyn_gather(k, ol)
            i_s = _dyn_gather(iv, ol)
            if p % 2 == 0:
                plsc.store_scatter(key_b, [dest], k_s)
                plsc.store_scatter(idx_b, [dest >> 7, dest & 127], i_s)
            else:
                plsc.store_scatter(key_a, [dest], k_s)
                plsc.store_scatter(idx_a, [dest], i_s)
            plsc.addupdate_scatter(hist, [sd], rank + 1, mask=is_end)
            return 0

        lax.fori_loop(0, CV, _place, 0)


    pltpu.sync_copy(key_b, shared.at[sid])
    plsc.subcore_barrier()
    for j in range(NT):
        @pl.when(j != q)
        def _copy(j=j):
            pltpu.sync_copy(shared.at[g * 4 + j], peers.at[j - (j > q)])

    def _rank(v, m):
        k = key_b[pl.ds(v * 16, 16)]
        acc = v * 16 + lane
        for jj in range(NT - 1):
            jj_v = jnp.full((16,), jj, jnp.int32)
            le = jj_v < q_v
            cnt = jnp.zeros((16,), jnp.int32)
            for bit in range(13, -1, -1):
                c2 = cnt + (1 << bit)
                safe = jnp.minimum(c2, C)
                x = plsc.load_gather(peers, [jj_v, safe - 1])
                take = (c2 <= C) & ((x < k) | ((x == k) & le))
                cnt = jnp.where(take, c2, cnt)
            acc = acc + cnt
        ranks[pl.ds(v * 16, 16)] = acc
        win = acc < K
        return m + jnp.max(plsc.all_reduce_population_count(win))

    m = lax.fori_loop(0, CV, _rank, jnp.int32(0))

    n_chunks = (m + 63) // 64
    b_node = b * N
    b_out = b * K

    def _chunk(c, _):
        for r8 in range(4):
            pos = c * 64 + r8 * 16 + lane
            up = jnp.minimum(pos, m - 1)
            nid = plsc.load_gather(idx_b, [up >> 7, up & 127])
            rk = plsc.load_gather(ranks, [up])
            idxc[0, pl.ds(r8 * 16, 16)] = nid + b_node
            rankc[0, pl.ds(r8 * 16, 16)] = rk + b_out
        pltpu.async_copy(h_hbm.at[idxc.at[0]], rows.at[pl.ds(0, 64)], sem).wait()
        for r8 in range(4):
            pos = c * 64 + r8 * 16 + lane
            up = jnp.minimum(pos, m - 1)
            kg = plsc.load_gather(key_b, [up])
            s16 = lax.bitcast_convert_type(~kg, jnp.float32)
            for j in range(16):
                sj = _dyn_gather(s16, jnp.full((16,), j, jnp.int32))
                row = r8 * 16 + j
                for f in range(8):
                    rows[row, pl.ds(f * 16, 16)] = (
                        rows[row, pl.ds(f * 16, 16)] * sj)
        pltpu.async_copy(rows.at[pl.ds(0, 64)], out_hbm.at[rankc.at[0]], sem).wait()
        return 0

    lax.fori_loop(0, n_chunks, _chunk, 0)


def kernel(h, W, b):
    bs, n, d = h.shape
    s = _scores(h, W, b)
    s_pad = jnp.pad(s, ((0, 0), (0, N_PAD - n)))
    s3d = s_pad.reshape(B * NT, CR, 128)
    h2 = h.reshape(bs * n, d)
    out = _sc_topk(s3d, h2)
    return out.reshape(bs, K, d)


# speedup vs baseline: 2.0267x; 1.6227x over previous
import functools

import jax
import jax.numpy as jnp
from jax import lax
from jax.experimental import pallas as pl
from jax.experimental.pallas import tpu as pltpu
from jax.experimental.pallas import tpu_sc as plsc

K_RATIO = 0.5

B = 8
N = 50000
D = 128
K = 25000
NT = 4
C = 12544
CV = C // 16
CR = C // 128
N_PAD = NT * C

_RADIX = ((0, 2047), (11, 2047), (22, 1023))


def _score_body(h_ref, w_ref, b_ref, s_ref):
    hb = h_ref[0]
    wt = w_ref[...]
    acc = lax.dot_general(wt, hb, (((1,), (1,)), ((), ())),
                          preferred_element_type=jnp.float32)
    s_ref[0, 0] = jax.nn.sigmoid(acc + b_ref[0])


def _scores(h, W, b):
    bs, n, d = h.shape
    blk = 2000
    s4 = pl.pallas_call(
        _score_body,
        grid=(bs, n // blk),
        in_specs=[
            pl.BlockSpec((1, blk, d), lambda i, j: (i, j, 0)),
            pl.BlockSpec((1, d), lambda i, j: (0, 0)),
            pl.BlockSpec(memory_space=pltpu.SMEM),
        ],
        out_specs=pl.BlockSpec((1, 1, 1, blk), lambda i, j: (i, j, 0, 0)),
        out_shape=jax.ShapeDtypeStruct((bs, n // blk, 1, blk), jnp.float32),
    )(h, W.reshape(1, d), b)
    return s4.reshape(bs, n)


def _dyn_gather(x, idx):
    dnums = lax.GatherDimensionNumbers(
        offset_dims=(), collapsed_slice_dims=(0,), start_index_map=(0,))
    return lax.gather(x, idx[:, None], dnums, (1,),
                      mode=lax.GatherScatterMode.PROMISE_IN_BOUNDS)


def _vreg_runs(d, lane):
    comp = d * 16 + lane
    csort, _ = plsc.sort_key_val(comp, lane)
    sd = csort >> 4
    ol = csort & 15
    prev = _dyn_gather(sd, jnp.maximum(lane - 1, 0))
    is_start = (lane == 0) | (sd != prev)
    startpos = plsc.cummax(jnp.where(is_start, lane, 0))
    rank = lane - startpos
    nxt = _dyn_gather(sd, jnp.minimum(lane + 1, 15))
    is_end = (lane == 15) | (sd != nxt)
    return sd, ol, rank, is_end


_sc_mesh = plsc.VectorSubcoreMesh(core_axis_name="c", subcore_axis_name="s")


@functools.partial(
    pl.kernel,
    out_type=jax.ShapeDtypeStruct((B * K, D), jnp.float32),
    mesh=_sc_mesh,
    scratch_types=[
        pltpu.VMEM((C,), jnp.int32),
        pltpu.VMEM((C,), jnp.int32),
        pltpu.VMEM((C,), jnp.int32),
        pltpu.VMEM((CR, 128), jnp.int32),
        pltpu.VMEM((4096,), jnp.int32),
        pltpu.VMEM((NT - 1, C), jnp.int32),
        pltpu.VMEM((3, 32, 128), jnp.float32),
        pltpu.VMEM((3, 32), jnp.int32),
        pltpu.VMEM((3, 32), jnp.int32),
        pltpu.VMEM_SHARED((16, C), jnp.int32),
        pltpu.SemaphoreType.DMA,
        pltpu.SemaphoreType.DMA,
        pltpu.SemaphoreType.DMA,
        pltpu.SemaphoreType.DMA,
        pltpu.SemaphoreType.DMA,
        pltpu.SemaphoreType.DMA,
    ],
    compiler_params=pltpu.CompilerParams(needs_layout_passes=False),
)
def _sc_topk(s_hbm, h_hbm, out_hbm, key_a, idx_a, key_b, idx_b, hist,
             peers, rows, idxc, rankc, shared, g0, g1, g2, s0, s1, s2):
    ranks = key_a
    cid = lax.axis_index("c")
    sid = lax.axis_index("s")
    b = cid * 4 + sid // 4
    q = sid % 4
    g = sid // 4
    lane = lax.iota(jnp.int32, 16)
    q_v = jnp.full((16,), 0, jnp.int32) + q

    pltpu.sync_copy(s_hbm.at[b * 4 + q, pl.ds(0, 32)], rows.at[0])
    pltpu.sync_copy(s_hbm.at[b * 4 + q, pl.ds(32, 32)], rows.at[1])
    pltpu.sync_copy(s_hbm.at[b * 4 + q, pl.ds(64, 32)], rows.at[2])

    def _build(v, _):
        r = v >> 3
        sv = rows[r >> 5, r & 31, pl.ds((v & 7) * 16, 16)]
        key_a[pl.ds(v * 16, 16)] = ~lax.bitcast_convert_type(sv, jnp.int32)
        idx_a[pl.ds(v * 16, 16)] = q * C + v * 16 + lane
        return 0

    lax.fori_loop(0, 96 * 8, _build, 0)
    pltpu.sync_copy(s_hbm.at[b * 4 + q, pl.ds(96, CR - 96)],
                    rows.at[0, pl.ds(0, CR - 96)])

    def _build2(v, _):
        r = (v >> 3) - 96
        sv = rows[0, r, pl.ds((v & 7) * 16, 16)]
        key_a[pl.ds(v * 16, 16)] = ~lax.bitcast_convert_type(sv, jnp.int32)
        idx_a[pl.ds(v * 16, 16)] = q * C + v * 16 + lane
        return 0

    lax.fori_loop(96 * 8, CV, _build2, 0)

    def _zero2(i, _):
        hist[pl.ds(i * 16, 16)] = jnp.zeros((16,), jnp.int32)
        return 0

    lax.fori_loop(0, 256, _zero2, 0)

    def _hist01(u, _):
        for h in range(2):
            v = u * 2 + h
            k = key_a[pl.ds(v * 16, 16)]
            for pp in range(2):
                d = (k >> _RADIX[pp][0]) & _RADIX[pp][1]
                sd, _, rank, is_end = _vreg_runs(d, lane)
                plsc.addupdate_scatter(
                    hist, [sd + pp * 2048], rank + 1, mask=is_end)
        return 0

    lax.fori_loop(0, CV // 2, _hist01, 0)

    for p, (shift, mask) in enumerate(_RADIX):
        hslot = min(p, 1)
        if p == 2:
            def _zero(i, _):
                hist[pl.ds(2048 + i * 16, 16)] = jnp.zeros((16,), jnp.int32)
                return 0

            lax.fori_loop(0, 128, _zero, 0)

            def _hist(u, _, shift=shift, mask=mask):
                for h in range(2):
                    v = u * 2 + h
                    k = key_b[pl.ds(v * 16, 16)]
                    d = (k >> shift) & mask
                    sd, _, rank, is_end = _vreg_runs(d, lane)
                    plsc.addupdate_scatter(
                        hist, [sd + 2048], rank + 1, mask=is_end)
                return 0

            lax.fori_loop(0, CV // 2, _hist, 0)

        def _scan(i, run, hslot=hslot):
            hv = hist[pl.ds(hslot * 2048 + i * 16, 16)]
            cs = plsc.cumsum(hv)
            hist[pl.ds(hslot * 2048 + i * 16, 16)] = cs - hv + run
            return run + jnp.max(cs)

        lax.fori_loop(0, 128, _scan, jnp.int32(0))

        def _place(u, _, shift=shift, mask=mask, p=p, hslot=hslot):
            ksrc = (key_a, key_b)[p % 2]
            for h in range(2):
                v = u * 2 + h
                k = ksrc[pl.ds(v * 16, 16)]
                if p % 2 == 0:
                    iv = idx_a[pl.ds(v * 16, 16)]
                else:
                    iv = idx_b[v >> 3, pl.ds((v & 7) * 16, 16)]
                d = (k >> shift) & mask
                sd, ol, rank, is_end = _vreg_runs(d, lane)
                base = plsc.load_gather(hist, [sd + hslot * 2048])
                dest = base + rank
                k_s = _dyn_gather(k, ol)
                i_s = _dyn_gather(iv, ol)
                if p % 2 == 0:
                    plsc.store_scatter(key_b, [dest], k_s)
                    plsc.store_scatter(idx_b, [dest >> 7, dest & 127], i_s)
                else:
                    plsc.store_scatter(key_a, [dest], k_s)
                    plsc.store_scatter(idx_a, [dest], i_s)
                plsc.addupdate_scatter(hist, [sd + hslot * 2048], rank + 1,
                                       mask=is_end)
            return 0

        lax.fori_loop(0, CV // 2, _place, 0)


    pltpu.sync_copy(key_b, shared.at[sid])
    plsc.subcore_barrier()
    for j in range(NT):
        @pl.when(j != q)
        def _copy(j=j):
            pltpu.sync_copy(shared.at[g * 4 + j], peers.at[j - (j > q)])

    def _rank(state):
        u, m, _ = state
        pcs = []
        for h in range(2):
            v = u * 2 + h
            k = key_b[pl.ds(v * 16, 16)]
            acc = v * 16 + lane
            for jj in range(NT - 1):
                jj_v = jnp.full((16,), jj, jnp.int32)
                le = jj_v < q_v
                cnt = jnp.zeros((16,), jnp.int32)
                for bit in range(13, -1, -1):
                    c2 = cnt + (1 << bit)
                    safe = jnp.minimum(c2, C)
                    x = plsc.load_gather(peers, [jj_v, safe - 1])
                    take = (c2 <= C) & ((x < k) | ((x == k) & le))
                    cnt = jnp.where(take, c2, cnt)
                acc = acc + cnt
            ranks[pl.ds(v * 16, 16)] = acc
            win = acc < K
            pcs.append(jnp.max(plsc.all_reduce_population_count(win)))
        pc = pcs[0] + pcs[1]
        return (u + 1, m + pc, pcs[1] > 0)

    _, m, _ = lax.while_loop(
        lambda st: (st[0] < CV // 2) & st[2], _rank,
        (jnp.int32(0), jnp.int32(0), jnp.bool_(True)))

    n_chunks = (m + 31) // 32
    b_node = b * N
    b_out = b * K

    def _build_idx(c, buf):
        for r8 in range(2):
            pos = c * 32 + r8 * 16 + lane
            up = jnp.minimum(pos, m - 1)
            nid = plsc.load_gather(idx_b, [up >> 7, up & 127])
            rk = plsc.load_gather(ranks, [up])
            idxc[buf, pl.ds(r8 * 16, 16)] = nid + b_node
            rankc[buf, pl.ds(r8 * 16, 16)] = rk + b_out

    def _gather(buf, gsem):
        return pltpu.make_async_copy(
            h_hbm.at[idxc.at[buf]], rows.at[buf], gsem)

    def _scatter(buf, ssem):
        return pltpu.make_async_copy(
            rows.at[buf], out_hbm.at[rankc.at[buf]], ssem)

    def _scale(c, buf):
        for r8 in range(2):
            pos = c * 32 + r8 * 16 + lane
            up = jnp.minimum(pos, m - 1)
            kg = plsc.load_gather(key_b, [up])
            s16 = lax.bitcast_convert_type(~kg, jnp.float32)
            for j in range(16):
                sj = _dyn_gather(s16, jnp.full((16,), j, jnp.int32))
                row = r8 * 16 + j
                for f in range(8):
                    rows[buf, row, pl.ds(f * 16, 16)] = (
                        rows[buf, row, pl.ds(f * 16, 16)] * sj)

    @pl.when(n_chunks > 0)
    def _pro0():
        _build_idx(0, 0)
        _gather(0, g0).start()

    @pl.when(n_chunks > 1)
    def _pro1():
        _build_idx(1, 1)
        _gather(1, g1).start()

    def _triple(t, _):
        ca = t * 3

        @pl.when(ca + 2 < n_chunks)
        def _():
            @pl.when(t >= 1)
            def _():
                _scatter(2, s2).wait()
            _build_idx(ca + 2, 2)
            _gather(2, g2).start()
        _gather(0, g0).wait()
        _scale(ca, 0)
        _scatter(0, s0).start()

        @pl.when(ca + 1 < n_chunks)
        def _():
            _gather(1, g1).wait()
            _scale(ca + 1, 1)
            _scatter(1, s1).start()

        @pl.when(ca + 3 < n_chunks)
        def _():
            _scatter(0, s0).wait()
            _build_idx(ca + 3, 0)
            _gather(0, g0).start()

        @pl.when(ca + 2 < n_chunks)
        def _():
            _gather(2, g2).wait()
            _scale(ca + 2, 2)
            _scatter(2, s2).start()

        @pl.when(ca + 4 < n_chunks)
        def _():
            _scatter(1, s1).wait()
            _build_idx(ca + 4, 1)
            _gather(1, g1).start()
        return 0

    lax.fori_loop(0, (n_chunks + 2) // 3, _triple, 0)

    @pl.when(n_chunks >= 1)
    def _d0():
        _scatter(0, s0).wait()

    @pl.when(n_chunks >= 2)
    def _d1():
        _scatter(1, s1).wait()

    @pl.when(n_chunks >= 3)
    def _d2():
        _scatter(2, s2).wait()


def kernel(h, W, b):
    bs, n, d = h.shape
    s = _scores(h, W, b)
    s_pad = jnp.pad(s, ((0, 0), (0, N_PAD - n)))
    s3d = s_pad.reshape(B * NT, CR, 128)
    h2 = h.reshape(bs * n, d)
    out = _sc_topk(s3d, h2)
    return out.reshape(bs, K, d)
